# baseline (speedup 1.0000x reference)
import functools

import jax
import jax.numpy as jnp
from jax import lax
from jax.experimental import pallas as pl
from jax.experimental.pallas import tpu as pltpu
from jax.experimental.pallas import tpu_sc as plsc

B = 1024
N = 100000

NC, NS, L = 2, 16, 16
NW = NC * NS
BPW = B // NW
NROWS16 = (B * N) // L


def _sc_gather_body(xflat_hbm, tgt_hbm, v_hbm, tgt_v, idx_v, val_v, sem):
    wid = lax.axis_index("s") * NC + lax.axis_index("c")
    base = wid * BPW
    pltpu.sync_copy(tgt_hbm.at[pl.ds(base, BPW)], tgt_v)
    for c in range(BPW // L):
        t = tgt_v[pl.ds(c * L, L)]
        row_id = base + c * L + lax.broadcasted_iota(jnp.int32, (L,), 0)
        idx_v[pl.ds(c * L, L)] = row_id * N + t
    pltpu.async_copy(xflat_hbm.at[idx_v], val_v, sem).wait()
    pltpu.sync_copy(val_v, v_hbm.at[pl.ds(base, BPW)])


def _make_sc_gather():
    return functools.partial(
        pl.kernel,
        mesh=plsc.VectorSubcoreMesh(core_axis_name="c", subcore_axis_name="s"),
        out_type=jax.ShapeDtypeStruct((B,), jnp.float32),
        scratch_types=[
            pltpu.VMEM((BPW,), jnp.int32),
            pltpu.VMEM((BPW,), jnp.int32),
            pltpu.VMEM((BPW,), jnp.float32),
            pltpu.SemaphoreType.DMA,
        ],
    )(_sc_gather_body)


SC_RPW = 32
SC_ROWS = NW * SC_RPW
B_TC = B - SC_ROWS
SC_BASE = B_TC
CHUNK = 10000
NCK = N // CHUNK
UNR = 5
INNER = CHUNK // (16 * UNR)

CH = 8
NBUF = 8
NCHUNK = B_TC // CH
GROUPS = NCHUNK // NBUF
SCALE = 100.0 / B


def _stream_body(x_hbm, v_ref, t_ref, rank_v, *bufs_sems):
    bufs = bufs_sems[:NBUF]
    sems = bufs_sems[NBUF:]

    def dma(b, c):
        return pltpu.make_async_copy(
            x_hbm.at[pl.ds(c * CH, CH), :], bufs[b], sems[b])

    for b in range(NBUF):
        dma(b, b).start()

    def group(g, carry):
        for b in range(NBUF):
            c = g * NBUF + b
            dma(b, c).wait()
            x = bufs[b][...]
            base = c * CH
            v = v_ref[pl.ds(base, CH), :]
            t = t_ref[pl.ds(base, CH), :]
            col = lax.broadcasted_iota(jnp.int32, (CH, N), 1)
            beats = (x > v) | ((x == v) & (col < t))
            rank_v[pl.ds(base, CH), :] = jnp.sum(
                jnp.where(beats, 1, 0), axis=1, keepdims=True)

            @pl.when(g < GROUPS - 1)
            def _next():
                dma(b, (g + 1) * NBUF + b).start()
        return carry

    lax.fori_loop(0, GROUPS, group, 0)


_stream = pl.pallas_call(
    _stream_body,
    in_specs=[
        pl.BlockSpec(memory_space=pl.ANY),
        pl.BlockSpec(memory_space=pltpu.VMEM),
        pl.BlockSpec(memory_space=pltpu.VMEM),
    ],
    out_specs=pl.BlockSpec(memory_space=pltpu.VMEM),
    out_shape=jax.ShapeDtypeStruct((B_TC, 1), jnp.int32),
    scratch_shapes=([pltpu.VMEM((CH, N), jnp.float32)] * NBUF
                    + [pltpu.SemaphoreType.DMA] * NBUF),
)


def _sc_count_body(xflat_hbm, t_hbm, part_hbm, idxb, vb_v, tb_v, xb0,
                   xb1, out_v, semg, sem0, sem1):
    wid = lax.axis_index("s") * NC + lax.axis_index("c")
    bufs = (xb0, xb1)
    sems = (sem0, sem1)
    iota = lax.broadcasted_iota(jnp.int32, (L,), 0)

    def dma(rr, k, p):
        grow = SC_BASE + wid * SC_RPW + rr
        return pltpu.make_async_copy(
            xflat_hbm.at[pl.ds(grow * N + k * CHUNK, CHUNK)],
            bufs[p], sems[p])

    dma(0, 0, 0).start()

    def row_body(r, carry):
        grow = SC_BASE + wid * SC_RPW + r
        idxb[...] = jnp.full((L,), grow, jnp.int32)
        pltpu.async_copy(t_hbm.at[idxb], tb_v, semg).wait()
        t_b = tb_v[...]
        idxb[...] = grow * N + t_b
        pltpu.async_copy(xflat_hbm.at[idxb], vb_v, semg).wait()
        v_b = vb_v[...]
        acc = jnp.zeros((L,), jnp.int32)
        for k in range(NCK):
            p = k % 2
            if k + 1 < NCK:
                dma(r, k + 1, (k + 1) % 2).start()
            else:
                @pl.when(r + 1 < SC_RPW)
                def _pre():
                    dma(r + 1, 0, 0).start()
            dma(r, k, p).wait()
            buf = bufs[p]
            col0 = k * CHUNK

            def it_body(i, a, buf=buf, col0=col0, v_b=v_b, t_b=t_b):
                for u in range(UNR):
                    off = i * (L * UNR) + u * L
                    xx = buf[pl.ds(off, L)]
                    col = col0 + off + iota
                    beats = (xx > v_b) | ((xx == v_b) & (col < t_b))
                    a = a + jnp.where(beats, 1, 0)
                return a

            acc = lax.fori_loop(0, INNER, it_body, acc)
        out_v[r] = acc
        return carry

    lax.fori_loop(0, SC_RPW, row_body, 0)
    pltpu.sync_copy(out_v, part_hbm.at[wid])


def _make_sc_count():
    return functools.partial(
        pl.kernel,
        mesh=plsc.VectorSubcoreMesh(core_axis_name="c", subcore_axis_name="s"),
        out_type=jax.ShapeDtypeStruct((NW, SC_RPW, L), jnp.int32),
        scratch_types=[
            pltpu.VMEM((L,), jnp.int32),
            pltpu.VMEM((L,), jnp.float32),
            pltpu.VMEM((L,), jnp.int32),
            pltpu.VMEM((CHUNK,), jnp.float32),
            pltpu.VMEM((CHUNK,), jnp.float32),
            pltpu.VMEM((SC_RPW, L), jnp.int32),
            pltpu.SemaphoreType.DMA,
            pltpu.SemaphoreType.DMA,
            pltpu.SemaphoreType.DMA,
        ],
    )(_sc_count_body)


def _final_body(part_ref, c1_ref, c5_ref):
    rank = jnp.sum(part_ref[...], axis=1, keepdims=True)
    c1_ref[0, 0] = jnp.sum(jnp.where(rank < 1, SCALE, 0.0))
    c5_ref[0, 0] = jnp.sum(jnp.where(rank < 5, SCALE, 0.0))


_final = pl.pallas_call(
    _final_body,
    in_specs=[pl.BlockSpec(memory_space=pltpu.VMEM)],
    out_specs=[
        pl.BlockSpec(memory_space=pltpu.SMEM),
        pl.BlockSpec(memory_space=pltpu.SMEM),
    ],
    out_shape=[jax.ShapeDtypeStruct((1, 1), jnp.float32)] * 2,
)


def kernel(outputs, targets):
    tgt = targets.astype(jnp.int32)
    xflat = outputs.reshape(B * N)
    part = _make_sc_count()(xflat, tgt)
    c1, c5 = _final(part.reshape(SC_ROWS, L))
    return (c1.reshape(1), c5.reshape(1))


# speedup vs baseline: 1.0062x; 1.0062x over previous
import functools

import jax
import jax.numpy as jnp
from jax import lax
from jax.experimental import pallas as pl
from jax.experimental.pallas import tpu as pltpu
from jax.experimental.pallas import tpu_sc as plsc

B = 1024
N = 100000

NC, NS, L = 2, 16, 16
NW = NC * NS
BPW = B // NW
NROWS16 = (B * N) // L


def _sc_gather_body(xflat_hbm, tgt_hbm, v_hbm, tgt_v, idx_v, val_v, sem):
    wid = lax.axis_index("s") * NC + lax.axis_index("c")
    base = wid * BPW
    pltpu.sync_copy(tgt_hbm.at[pl.ds(base, BPW)], tgt_v)
    for c in range(BPW // L):
        t = tgt_v[pl.ds(c * L, L)]
        row_id = base + c * L + lax.broadcasted_iota(jnp.int32, (L,), 0)
        idx_v[pl.ds(c * L, L)] = row_id * N + t
    pltpu.async_copy(xflat_hbm.at[idx_v], val_v, sem).wait()
    pltpu.sync_copy(val_v, v_hbm.at[pl.ds(base, BPW)])


def _make_sc_gather():
    return functools.partial(
        pl.kernel,
        mesh=plsc.VectorSubcoreMesh(core_axis_name="c", subcore_axis_name="s"),
        out_type=jax.ShapeDtypeStruct((B,), jnp.float32),
        scratch_types=[
            pltpu.VMEM((BPW,), jnp.int32),
            pltpu.VMEM((BPW,), jnp.int32),
            pltpu.VMEM((BPW,), jnp.float32),
            pltpu.SemaphoreType.DMA,
        ],
    )(_sc_gather_body)


SC_RPW = 32
SC_ROWS = NW * SC_RPW
B_TC = B - SC_ROWS
SC_BASE = B_TC
CHUNK = 50000
NCK = N // CHUNK
UNR = 5
INNER = CHUNK // (16 * UNR)

CH = 8
NBUF = 8
NCHUNK = B_TC // CH
GROUPS = NCHUNK // NBUF
SCALE = 100.0 / B


def _stream_body(x_hbm, v_ref, t_ref, rank_v, *bufs_sems):
    bufs = bufs_sems[:NBUF]
    sems = bufs_sems[NBUF:]

    def dma(b, c):
        return pltpu.make_async_copy(
            x_hbm.at[pl.ds(c * CH, CH), :], bufs[b], sems[b])

    for b in range(NBUF):
        dma(b, b).start()

    def group(g, carry):
        for b in range(NBUF):
            c = g * NBUF + b
            dma(b, c).wait()
            x = bufs[b][...]
            base = c * CH
            v = v_ref[pl.ds(base, CH), :]
            t = t_ref[pl.ds(base, CH), :]
            col = lax.broadcasted_iota(jnp.int32, (CH, N), 1)
            beats = (x > v) | ((x == v) & (col < t))
            rank_v[pl.ds(base, CH), :] = jnp.sum(
                jnp.where(beats, 1, 0), axis=1, keepdims=True)

            @pl.when(g < GROUPS - 1)
            def _next():
                dma(b, (g + 1) * NBUF + b).start()
        return carry

    lax.fori_loop(0, GROUPS, group, 0)


_stream = pl.pallas_call(
    _stream_body,
    in_specs=[
        pl.BlockSpec(memory_space=pl.ANY),
        pl.BlockSpec(memory_space=pltpu.VMEM),
        pl.BlockSpec(memory_space=pltpu.VMEM),
    ],
    out_specs=pl.BlockSpec(memory_space=pltpu.VMEM),
    out_shape=jax.ShapeDtypeStruct((B_TC, 1), jnp.int32),
    scratch_shapes=([pltpu.VMEM((CH, N), jnp.float32)] * NBUF
                    + [pltpu.SemaphoreType.DMA] * NBUF),
)


def _sc_count_body(xflat_hbm, t_hbm, part_hbm, idxb, vb_v, tb_v, xb0,
                   xb1, out_v, semg, sem0, sem1):
    wid = lax.axis_index("s") * NC + lax.axis_index("c")
    bufs = (xb0, xb1)
    sems = (sem0, sem1)
    iota = lax.broadcasted_iota(jnp.int32, (L,), 0)

    def dma(rr, k, p):
        grow = SC_BASE + wid * SC_RPW + rr
        return pltpu.make_async_copy(
            xflat_hbm.at[pl.ds(grow * N + k * CHUNK, CHUNK)],
            bufs[p], sems[p])

    dma(0, 0, 0).start()

    def row_body(r, carry):
        grow = SC_BASE + wid * SC_RPW + r
        idxb[...] = jnp.full((L,), grow, jnp.int32)
        pltpu.async_copy(t_hbm.at[idxb], tb_v, semg).wait()
        t_b = tb_v[...]
        idxb[...] = grow * N + t_b
        pltpu.async_copy(xflat_hbm.at[idxb], vb_v, semg).wait()
        v_b = vb_v[...]
        acc = jnp.zeros((L,), jnp.int32)
        for k in range(NCK):
            p = k % 2
            if k + 1 < NCK:
                dma(r, k + 1, (k + 1) % 2).start()
            else:
                @pl.when(r + 1 < SC_RPW)
                def _pre():
                    dma(r + 1, 0, 0).start()
            dma(r, k, p).wait()
            buf = bufs[p]
            col0 = k * CHUNK

            def it_body(i, a, buf=buf, col0=col0, v_b=v_b, t_b=t_b):
                for u in range(UNR):
                    off = i * (L * UNR) + u * L
                    xx = buf[pl.ds(off, L)]
                    col = col0 + off + iota
                    beats = (xx > v_b) | ((xx == v_b) & (col < t_b))
                    a = a + jnp.where(beats, 1, 0)
                return a

            acc = lax.fori_loop(0, INNER, it_body, acc)
        out_v[r] = acc
        return carry

    lax.fori_loop(0, SC_RPW, row_body, 0)
    pltpu.sync_copy(out_v, part_hbm.at[wid])


def _make_sc_count():
    return functools.partial(
        pl.kernel,
        mesh=plsc.VectorSubcoreMesh(core_axis_name="c", subcore_axis_name="s"),
        out_type=jax.ShapeDtypeStruct((NW, SC_RPW, L), jnp.int32),
        scratch_types=[
            pltpu.VMEM((L,), jnp.int32),
            pltpu.VMEM((L,), jnp.float32),
            pltpu.VMEM((L,), jnp.int32),
            pltpu.VMEM((CHUNK,), jnp.float32),
            pltpu.VMEM((CHUNK,), jnp.float32),
            pltpu.VMEM((SC_RPW, L), jnp.int32),
            pltpu.SemaphoreType.DMA,
            pltpu.SemaphoreType.DMA,
            pltpu.SemaphoreType.DMA,
        ],
    )(_sc_count_body)


def _final_body(part_ref, c1_ref, c5_ref):
    rank = jnp.sum(part_ref[...], axis=1, keepdims=True)
    c1_ref[0, 0] = jnp.sum(jnp.where(rank < 1, SCALE, 0.0))
    c5_ref[0, 0] = jnp.sum(jnp.where(rank < 5, SCALE, 0.0))


_final = pl.pallas_call(
    _final_body,
    in_specs=[pl.BlockSpec(memory_space=pltpu.VMEM)],
    out_specs=[
        pl.BlockSpec(memory_space=pltpu.SMEM),
        pl.BlockSpec(memory_space=pltpu.SMEM),
    ],
    out_shape=[jax.ShapeDtypeStruct((1, 1), jnp.float32)] * 2,
)


def kernel(outputs, targets):
    tgt = targets.astype(jnp.int32)
    xflat = outputs.reshape(B * N)
    part = _make_sc_count()(xflat, tgt)
    c1, c5 = _final(part.reshape(SC_ROWS, L))
    return (c1.reshape(1), c5.reshape(1))


# speedup vs baseline: 1.1566x; 1.1495x over previous
import functools

import jax
import jax.numpy as jnp
from jax import lax
from jax.experimental import pallas as pl
from jax.experimental.pallas import tpu as pltpu
from jax.experimental.pallas import tpu_sc as plsc

B = 1024
N = 100000

NC, NS, L = 2, 16, 16
NW = NC * NS
BPW = B // NW
NROWS16 = (B * N) // L


def _sc_gather_body(xflat_hbm, tgt_hbm, v_hbm, tgt_v, idx_v, val_v, sem):
    wid = lax.axis_index("s") * NC + lax.axis_index("c")
    base = wid * BPW
    pltpu.sync_copy(tgt_hbm.at[pl.ds(base, BPW)], tgt_v)
    for c in range(BPW // L):
        t = tgt_v[pl.ds(c * L, L)]
        row_id = base + c * L + lax.broadcasted_iota(jnp.int32, (L,), 0)
        idx_v[pl.ds(c * L, L)] = row_id * N + t
    pltpu.async_copy(xflat_hbm.at[idx_v], val_v, sem).wait()
    pltpu.sync_copy(val_v, v_hbm.at[pl.ds(base, BPW)])


def _make_sc_gather():
    return functools.partial(
        pl.kernel,
        mesh=plsc.VectorSubcoreMesh(core_axis_name="c", subcore_axis_name="s"),
        out_type=jax.ShapeDtypeStruct((B,), jnp.float32),
        scratch_types=[
            pltpu.VMEM((BPW,), jnp.int32),
            pltpu.VMEM((BPW,), jnp.int32),
            pltpu.VMEM((BPW,), jnp.float32),
            pltpu.SemaphoreType.DMA,
        ],
    )(_sc_gather_body)


CH = 16
NBUF = 8
NCHUNK = B // CH
GROUPS = NCHUNK // NBUF
SCALE = 100.0 / B


def _stream_body(x_hbm, v_ref, t_ref, c1_ref, c5_ref, rank_v, *bufs_sems):
    bufs = bufs_sems[:NBUF]
    sems = bufs_sems[NBUF:]

    def dma(b, c):
        return pltpu.make_async_copy(
            x_hbm.at[pl.ds(c * CH, CH), :], bufs[b], sems[b])

    for b in range(NBUF):
        dma(b, b).start()

    def group(g, carry):
        for b in range(NBUF):
            c = g * NBUF + b
            dma(b, c).wait()
            x = bufs[b][...]
            base = c * CH
            v = v_ref[pl.ds(base, CH), :]
            t = t_ref[pl.ds(base, CH), :]
            col = lax.broadcasted_iota(jnp.int32, (CH, N), 1)
            beats = (x > v) | ((x == v) & (col < t))
            rank_v[pl.ds(base, CH), :] = jnp.sum(
                jnp.where(beats, 1, 0), axis=1, keepdims=True)

            @pl.when(g < GROUPS - 1)
            def _next():
                dma(b, (g + 1) * NBUF + b).start()
        return carry

    lax.fori_loop(0, GROUPS, group, 0)
    rank = rank_v[...]
    c1_ref[0, 0] = jnp.sum(jnp.where(rank < 1, SCALE, 0.0))
    c5_ref[0, 0] = jnp.sum(jnp.where(rank < 5, SCALE, 0.0))


_stream = pl.pallas_call(
    _stream_body,
    in_specs=[
        pl.BlockSpec(memory_space=pl.ANY),
        pl.BlockSpec(memory_space=pltpu.VMEM),
        pl.BlockSpec(memory_space=pltpu.VMEM),
    ],
    out_specs=[
        pl.BlockSpec(memory_space=pltpu.SMEM),
        pl.BlockSpec(memory_space=pltpu.SMEM),
    ],
    out_shape=[jax.ShapeDtypeStruct((1, 1), jnp.float32)] * 2,
    scratch_shapes=([pltpu.VMEM((B, 1), jnp.int32)]
                    + [pltpu.VMEM((CH, N), jnp.float32)] * NBUF
                    + [pltpu.SemaphoreType.DMA] * NBUF),
)


def kernel(outputs, targets):
    tgt = targets.astype(jnp.int32)
    xflat = outputs.reshape(B * N)
    v = _make_sc_gather()(xflat, tgt)
    c1, c5 = _stream(outputs, v.reshape(B, 1), tgt.reshape(B, 1))
    return (c1.reshape(1), c5.reshape(1))
